# Initial kernel scaffold; baseline (speedup 1.0000x reference)
import jax
import jax.numpy as jnp
from jax.experimental import pallas as pl


def kernel(signal, synapses_weight, coord0, coord1):
    raise NotImplementedError("write your pallas kernel here")


# speedup vs baseline: 265.7329x; 265.7329x over previous
import functools
import math

import jax
import jax.numpy as jnp
from jax import lax
from jax.experimental import pallas as pl
from jax.experimental.pallas import tpu as pltpu
from jax.experimental.pallas import tpu_sc as plsc

H = 512
W = 512
HW = H * W
S = 4
K = 9
R = S * K
B = 2

_PI = math.pi

_NC = 2
_NS = 16
_NW = _NC * _NS
_CHUNK = HW // _NW
_LANES = 16

_DELTAS = [(d0, d1) for d0 in (-1, 0, 1) for d1 in (-1, 0, 1)]


def _decard(angle):
    s = jnp.sin(angle)
    a = jnp.where(angle <= jnp.float32(0.5 * _PI), angle, jnp.float32(_PI) - angle)
    b = jnp.cos(angle)
    c = jnp.sqrt((1.0 + 1e-06) - s * s)
    f = a * b / c
    f = 2.0 * f / _PI
    f = f + 1.0
    f = f / 2.0
    f = f * (W - 1)
    return f


def _precompute_body(c0_ref, c1_ref, sw_ref, idx_ref, coef_ref):
    for s in range(S):
        c0 = c0_ref[s]
        c1 = c1_ref[s]
        w = sw_ref[s]
        dec0 = _decard(c0)
        dec1 = _decard(c1)
        var0 = []
        var1 = []
        for d in (-1, 0, 1):
            off = _PI * d / (W - 1)
            var0.append(jnp.round(_decard(c0 + off)))
            var1.append(jnp.round(_decard(c1 + off)))
        for k, (d0, d1) in enumerate(_DELTAS):
            v0 = var0[d0 + 1]
            v1 = var1[d1 + 1]
            dist = jnp.sqrt((v0 - dec0) ** 2 + (v1 - dec1) ** 2)
            sig = 6.0 * (1.0 - 2.0 * dist) / 1.0
            sig = 1.0 / (1.0 + jnp.exp(-sig))
            r = s * K + k
            idx_ref[r] = (v0 * float(W) + v1).astype(jnp.int32)
            coef_ref[r] = w * sig


_ROWS_BLK = 32


def _precompute(c0, c1, sw):
    grid = (H // _ROWS_BLK,)
    in_spec = pl.BlockSpec((S, _ROWS_BLK, W), lambda i: (0, i, 0))
    out_spec = pl.BlockSpec((R, _ROWS_BLK, W), lambda i: (0, i, 0))
    return pl.pallas_call(
        _precompute_body,
        grid=grid,
        in_specs=[in_spec, in_spec, in_spec],
        out_specs=[out_spec, out_spec],
        out_shape=[
            jax.ShapeDtypeStruct((R, H, W), jnp.int32),
            jax.ShapeDtypeStruct((R, H, W), jnp.float32),
        ],
    )(c0, c1, sw)


def _sc_body(t0_ref, t1_ref, idx_ref, coef_ref, out_ref,
             idx_v, v0_v, v1_v, coef_v, acc0_v, acc1_v, sem0, sem1):
    wid = lax.axis_index("s") * _NC + lax.axis_index("c")
    base = wid * _CHUNK
    zero_f = jnp.zeros((_LANES,), jnp.float32)

    def zero_body(j, carry):
        p = j * _LANES
        acc0_v[pl.ds(p, _LANES)] = zero_f
        acc1_v[pl.ds(p, _LANES)] = zero_f
        return carry

    lax.fori_loop(0, _CHUNK // _LANES, zero_body, 0)

    def row_body(r, carry):
        off = r * HW + base
        pltpu.sync_copy(idx_ref.at[pl.ds(off, _CHUNK)], idx_v)
        pltpu.sync_copy(coef_ref.at[pl.ds(off, _CHUNK)], coef_v)
        cp0 = pltpu.async_copy(t0_ref.at[idx_v], v0_v, sem0)
        cp1 = pltpu.async_copy(t1_ref.at[idx_v], v1_v, sem1)
        cp0.wait()
        cp1.wait()

        def mac_body(j, c2):
            p = j * _LANES
            sl = pl.ds(p, _LANES)
            cf = coef_v[sl]
            acc0_v[sl] = acc0_v[sl] + v0_v[sl] * cf
            acc1_v[sl] = acc1_v[sl] + v1_v[sl] * cf
            return c2

        lax.fori_loop(0, _CHUNK // _LANES, mac_body, 0)
        return carry

    lax.fori_loop(0, R, row_body, 0)

    pltpu.sync_copy(acc0_v, out_ref.at[pl.ds(base, _CHUNK)])
    pltpu.sync_copy(acc1_v, out_ref.at[pl.ds(HW + base, _CHUNK)])


@functools.cache
def _sc_gather_mac():
    return pl.kernel(
        _sc_body,
        mesh=plsc.VectorSubcoreMesh(
            core_axis_name="c", subcore_axis_name="s", num_cores=_NC
        ),
        out_type=jax.ShapeDtypeStruct((B * HW,), jnp.float32),
        scratch_types=[
            pltpu.VMEM((_CHUNK,), jnp.int32),
            pltpu.VMEM((_CHUNK,), jnp.float32),
            pltpu.VMEM((_CHUNK,), jnp.float32),
            pltpu.VMEM((_CHUNK,), jnp.float32),
            pltpu.VMEM((_CHUNK,), jnp.float32),
            pltpu.VMEM((_CHUNK,), jnp.float32),
            pltpu.SemaphoreType.DMA,
            pltpu.SemaphoreType.DMA,
        ],
    )


def kernel(signal, synapses_weight, coord0, coord1):
    idx, coef = _precompute(coord0, coord1, synapses_weight)
    sig = signal.reshape(B, HW)
    out = _sc_gather_mac()(sig[0], sig[1], idx.reshape(R * HW), coef.reshape(R * HW))
    return out.reshape(B, H, W)


# speedup vs baseline: 269.9719x; 1.0160x over previous
import functools
import math

import jax
import jax.numpy as jnp
from jax import lax
from jax.experimental import pallas as pl
from jax.experimental.pallas import tpu as pltpu
from jax.experimental.pallas import tpu_sc as plsc

H = 512
W = 512
HW = H * W
S = 4
K = 9
R = S * K
B = 2

_PI = math.pi

_NC = 2
_NS = 16
_NW = _NC * _NS
_CHUNK = HW // _NW
_LANES = 16

_DELTAS = [(d0, d1) for d0 in (-1, 0, 1) for d1 in (-1, 0, 1)]


def _decard(angle):
    s = jnp.sin(angle)
    a = jnp.where(angle <= jnp.float32(0.5 * _PI), angle, jnp.float32(_PI) - angle)
    b = jnp.cos(angle)
    c = jnp.sqrt((1.0 + 1e-06) - s * s)
    f = a * b / c
    f = 2.0 * f / _PI
    f = f + 1.0
    f = f / 2.0
    f = f * (W - 1)
    return f


def _precompute_body(c0_ref, c1_ref, sw_ref, idx_ref, coef_ref):
    for s in range(S):
        c0 = c0_ref[s]
        c1 = c1_ref[s]
        w = sw_ref[s]
        dec0 = _decard(c0)
        dec1 = _decard(c1)
        var0 = []
        var1 = []
        for d in (-1, 0, 1):
            off = _PI * d / (W - 1)
            var0.append(jnp.round(_decard(c0 + off)))
            var1.append(jnp.round(_decard(c1 + off)))
        for k, (d0, d1) in enumerate(_DELTAS):
            v0 = var0[d0 + 1]
            v1 = var1[d1 + 1]
            dist = jnp.sqrt((v0 - dec0) ** 2 + (v1 - dec1) ** 2)
            sig = 6.0 * (1.0 - 2.0 * dist) / 1.0
            sig = 1.0 / (1.0 + jnp.exp(-sig))
            r = s * K + k
            idx_ref[r] = (v0 * float(W) + v1).astype(jnp.int32)
            coef_ref[r] = w * sig


_ROWS_BLK = 32


def _precompute(c0, c1, sw):
    grid = (H // _ROWS_BLK,)
    in_spec = pl.BlockSpec((S, _ROWS_BLK, W), lambda i: (0, i, 0))
    out_spec = pl.BlockSpec((R, _ROWS_BLK, W), lambda i: (0, i, 0))
    return pl.pallas_call(
        _precompute_body,
        grid=grid,
        in_specs=[in_spec, in_spec, in_spec],
        out_specs=[out_spec, out_spec],
        out_shape=[
            jax.ShapeDtypeStruct((R, H, W), jnp.int32),
            jax.ShapeDtypeStruct((R, H, W), jnp.float32),
        ],
    )(c0, c1, sw)


def _sc_body(t0_ref, t1_ref, idx_ref, coef_ref, out_ref,
             idx0_v, idx1_v, v00_v, v10_v, v01_v, v11_v, cf0_v, cf1_v,
             acc0_v, acc1_v,
             si0, sc0, sg00, sg10, si1, sc1, sg01, sg11):
    wid = lax.axis_index("s") * _NC + lax.axis_index("c")
    base = wid * _CHUNK
    zero_f = jnp.zeros((_LANES,), jnp.float32)

    idx_b = (idx0_v, idx1_v)
    v0_b = (v00_v, v01_v)
    v1_b = (v10_v, v11_v)
    cf_b = (cf0_v, cf1_v)
    sem_i = (si0, si1)
    sem_c = (sc0, sc1)
    sem_g0 = (sg00, sg01)
    sem_g1 = (sg10, sg11)

    def zero_body(j, carry):
        p = j * _LANES
        acc0_v[pl.ds(p, _LANES)] = zero_f
        acc1_v[pl.ds(p, _LANES)] = zero_f
        return carry

    lax.fori_loop(0, _CHUNK // _LANES, zero_body, 0)

    def issue_ic(r, bi):
        off = r * HW + base
        cpi = pltpu.async_copy(idx_ref.at[pl.ds(off, _CHUNK)], idx_b[bi], sem_i[bi])
        cpc = pltpu.async_copy(coef_ref.at[pl.ds(off, _CHUNK)], cf_b[bi], sem_c[bi])
        return cpi, cpc

    def issue_g(bi):
        g0 = pltpu.async_copy(t0_ref.at[idx_b[bi]], v0_b[bi], sem_g0[bi])
        g1 = pltpu.async_copy(t1_ref.at[idx_b[bi]], v1_b[bi], sem_g1[bi])
        return g0, g1

    def mac(bi):
        def mac_body(j, c2):
            p = j * _LANES
            sl = pl.ds(p, _LANES)
            cf = cf_b[bi][sl]
            acc0_v[sl] = acc0_v[sl] + v0_b[bi][sl] * cf
            acc1_v[sl] = acc1_v[sl] + v1_b[bi][sl] * cf
            return c2

        lax.fori_loop(0, _CHUNK // _LANES, mac_body, 0)

    ic = issue_ic(0, 0)
    ic[0].wait()
    ic[1].wait()
    g_prev = issue_g(0)
    ic_next = issue_ic(1, 1)
    for r in range(1, R):
        bi = r % 2
        ic_next[0].wait()
        ic_next[1].wait()
        g_new = issue_g(bi)
        g_prev[0].wait()
        g_prev[1].wait()
        mac(1 - bi)
        if r + 1 < R:
            ic_next = issue_ic(r + 1, 1 - bi)
        g_prev = g_new
    g_prev[0].wait()
    g_prev[1].wait()
    mac((R - 1) % 2)

    pltpu.sync_copy(acc0_v, out_ref.at[pl.ds(base, _CHUNK)])
    pltpu.sync_copy(acc1_v, out_ref.at[pl.ds(HW + base, _CHUNK)])


@functools.cache
def _sc_gather_mac():
    return pl.kernel(
        _sc_body,
        mesh=plsc.VectorSubcoreMesh(
            core_axis_name="c", subcore_axis_name="s", num_cores=_NC
        ),
        out_type=jax.ShapeDtypeStruct((B * HW,), jnp.float32),
        scratch_types=[
            pltpu.VMEM((_CHUNK,), jnp.int32),
            pltpu.VMEM((_CHUNK,), jnp.int32),
            pltpu.VMEM((_CHUNK,), jnp.float32),
            pltpu.VMEM((_CHUNK,), jnp.float32),
            pltpu.VMEM((_CHUNK,), jnp.float32),
            pltpu.VMEM((_CHUNK,), jnp.float32),
            pltpu.VMEM((_CHUNK,), jnp.float32),
            pltpu.VMEM((_CHUNK,), jnp.float32),
            pltpu.VMEM((_CHUNK,), jnp.float32),
            pltpu.VMEM((_CHUNK,), jnp.float32),
        ] + [pltpu.SemaphoreType.DMA] * 8,
    )


def kernel(signal, synapses_weight, coord0, coord1):
    idx, coef = _precompute(coord0, coord1, synapses_weight)
    sig = signal.reshape(B, HW)
    out = _sc_gather_mac()(sig[0], sig[1], idx.reshape(R * HW), coef.reshape(R * HW))
    return out.reshape(B, H, W)
